# baseline (speedup 1.0000x reference)
import jax
import jax.numpy as jnp
from jax.experimental import pallas as pl

_N = 4096
_D = 256
_C = 32
_R = 64
_BM = 1024
_MAXI = 0x7FFFFFFF


def _metrics_block(dblk_ref, dt_ref, sqc_ref, crow_ref, limr_ref, ccol_ref,
                   vec_ref, num_ref):
    blk = pl.program_id(0)
    f32 = jnp.float32
    i32 = jnp.int32

    dot = jnp.dot(dblk_ref[...], dt_ref[...], preferred_element_type=f32)
    key = sqc_ref[...] - 2.0 * dot
    colids = jax.lax.broadcasted_iota(i32, (_BM, _N), 1)
    rowids = jax.lax.broadcasted_iota(i32, (_BM, _N), 0) + blk * _BM

    crow = crow_ref[...]
    limr = limr_ref[...]
    eqi = (crow == ccol_ref[...]).astype(i32)

    u = jax.lax.bitcast_convert_type(key, i32)
    s = jnp.where(u < 0, u ^ _MAXI, u)
    p = (s & ~1) | eqi
    p = jnp.where(colids == rowids, _MAXI, p)

    def step(t, carry):
        shift, cnt, r1, r5, r10, mp, numr = carry
        m = jnp.min(p - shift, axis=1, keepdims=True) + shift
        hit = (m & 1).astype(f32)
        cnt = cnt + hit
        r1 = r1 + jnp.where(t == 0, cnt, 0.0)
        r5 = r5 + jnp.where(t == 4, cnt, 0.0)
        r10 = r10 + jnp.where(t == 9, cnt, 0.0)
        mp = mp + (cnt / (t + 1).astype(f32)) * hit
        numr = numr + jnp.where(t.astype(f32) < limr, hit, 0.0)
        return m + (1 - 2**31), cnt, r1, r5, r10, mp, numr

    zb = jnp.zeros((_BM, 1), f32)
    init = (jnp.zeros((_BM, 1), i32), zb, zb, zb, zb, zb, zb)
    _, _, r1, r5, r10, mp, numr = jax.lax.fori_loop(0, _R, step, init)

    cls_lane = jax.lax.broadcasted_iota(i32, (_BM, _C), 1).astype(f32)
    ohrow = (crow == cls_lane).astype(f32)
    numpart = jax.lax.dot_general(numr, ohrow, (((0,), (0,)), ((), ())),
                                  preferred_element_type=f32)

    lane = jax.lax.broadcasted_iota(i32, (1, 128), 1)
    vecpart = (jnp.where(lane == 0, jnp.sum(r1), 0.0)
               + jnp.where(lane == 1, jnp.sum(r5) / 5.0, 0.0)
               + jnp.where(lane == 2, jnp.sum(r10) / 10.0, 0.0)
               + jnp.where(lane == 3, jnp.sum(mp), 0.0))

    @pl.when(blk == 0)
    def _init():
        vec_ref[...] = jnp.zeros_like(vec_ref)
        num_ref[...] = jnp.zeros_like(num_ref)

    vec_ref[...] += vecpart
    num_ref[...] += numpart


def kernel(d, c):
    f32 = jnp.float32
    cf = c.astype(f32)
    dt = d.T
    sqc = jnp.sum(d * d, axis=1)[None, :] + 128.0
    crow = cf[:, None]
    ccol = cf[None, :]
    counts = jnp.sum(cf[:, None] == jnp.arange(_C, dtype=f32)[None, :],
                     axis=0)
    lim = jnp.minimum(counts, f32(_R))
    limr = lim[c][:, None]

    vec, num = pl.pallas_call(
        _metrics_block,
        grid=(_N // _BM,),
        in_specs=[
            pl.BlockSpec((_BM, _D), lambda i: (i, 0)),
            pl.BlockSpec((_D, _N), lambda i: (0, 0)),
            pl.BlockSpec((1, _N), lambda i: (0, 0)),
            pl.BlockSpec((_BM, 1), lambda i: (i, 0)),
            pl.BlockSpec((_BM, 1), lambda i: (i, 0)),
            pl.BlockSpec((1, _N), lambda i: (0, 0)),
        ],
        out_specs=[
            pl.BlockSpec((1, 128), lambda i: (0, 0)),
            pl.BlockSpec((1, _C), lambda i: (0, 0)),
        ],
        out_shape=[
            jax.ShapeDtypeStruct((1, 128), f32),
            jax.ShapeDtypeStruct((1, _C), f32),
        ],
    )(d, dt, sqc, crow, limr, ccol)

    n = f32(_N)
    recalls = [vec[0, 0] / n, vec[0, 1] / n, vec[0, 2] / n]
    mapr = vec[0, 3] / (n * _R)
    den = jnp.maximum(counts * lim, 1.0)
    r_precision = jnp.mean(num[0] / den)
    return jnp.stack(recalls + [mapr, r_precision])


# speedup vs baseline: 1.0105x; 1.0105x over previous
import jax
import jax.numpy as jnp
from jax.experimental import pallas as pl

_N = 4096
_D = 256
_C = 32
_R = 64
_BM = 512
_MAXI = 0x7FFFFFFF


def _metrics_block(dblk_ref, dt_ref, sqc_ref, crow_ref, limr_ref, ccol_ref,
                   vec_ref, num_ref):
    blk = pl.program_id(0)
    f32 = jnp.float32
    i32 = jnp.int32

    dot = jnp.dot(dblk_ref[...], dt_ref[...], preferred_element_type=f32)
    key = sqc_ref[...] - 2.0 * dot
    colids = jax.lax.broadcasted_iota(i32, (_BM, _N), 1)
    rowids = jax.lax.broadcasted_iota(i32, (_BM, _N), 0) + blk * _BM

    crow = crow_ref[...]
    limr = limr_ref[...]
    eqi = (crow == ccol_ref[...]).astype(i32)

    u = jax.lax.bitcast_convert_type(key, i32)
    s = jnp.where(u < 0, u ^ _MAXI, u)
    p = (s & ~1) | eqi
    p = jnp.where(colids == rowids, _MAXI, p)

    def step(t, carry):
        shift, cnt, r1, r5, r10, mp, numr = carry
        m = jnp.min(p - shift, axis=1, keepdims=True) + shift
        hit = (m & 1).astype(f32)
        cnt = cnt + hit
        r1 = r1 + jnp.where(t == 0, cnt, 0.0)
        r5 = r5 + jnp.where(t == 4, cnt, 0.0)
        r10 = r10 + jnp.where(t == 9, cnt, 0.0)
        mp = mp + (cnt / (t + 1).astype(f32)) * hit
        numr = numr + jnp.where(t.astype(f32) < limr, hit, 0.0)
        return m + (1 - 2**31), cnt, r1, r5, r10, mp, numr

    zb = jnp.zeros((_BM, 1), f32)
    init = (jnp.zeros((_BM, 1), i32), zb, zb, zb, zb, zb, zb)
    _, _, r1, r5, r10, mp, numr = jax.lax.fori_loop(0, _R, step, init)

    cls_lane = jax.lax.broadcasted_iota(i32, (_BM, _C), 1).astype(f32)
    ohrow = (crow == cls_lane).astype(f32)
    numpart = jax.lax.dot_general(numr, ohrow, (((0,), (0,)), ((), ())),
                                  preferred_element_type=f32)

    lane = jax.lax.broadcasted_iota(i32, (1, 128), 1)
    vecpart = (jnp.where(lane == 0, jnp.sum(r1), 0.0)
               + jnp.where(lane == 1, jnp.sum(r5) / 5.0, 0.0)
               + jnp.where(lane == 2, jnp.sum(r10) / 10.0, 0.0)
               + jnp.where(lane == 3, jnp.sum(mp), 0.0))

    @pl.when(blk == 0)
    def _init():
        vec_ref[...] = jnp.zeros_like(vec_ref)
        num_ref[...] = jnp.zeros_like(num_ref)

    vec_ref[...] += vecpart
    num_ref[...] += numpart


def kernel(d, c):
    f32 = jnp.float32
    cf = c.astype(f32)
    dt = d.T
    sqc = jnp.sum(d * d, axis=1)[None, :] + 128.0
    crow = cf[:, None]
    ccol = cf[None, :]
    counts = jnp.sum(cf[:, None] == jnp.arange(_C, dtype=f32)[None, :],
                     axis=0)
    lim = jnp.minimum(counts, f32(_R))
    limr = lim[c][:, None]

    vec, num = pl.pallas_call(
        _metrics_block,
        grid=(_N // _BM,),
        in_specs=[
            pl.BlockSpec((_BM, _D), lambda i: (i, 0)),
            pl.BlockSpec((_D, _N), lambda i: (0, 0)),
            pl.BlockSpec((1, _N), lambda i: (0, 0)),
            pl.BlockSpec((_BM, 1), lambda i: (i, 0)),
            pl.BlockSpec((_BM, 1), lambda i: (i, 0)),
            pl.BlockSpec((1, _N), lambda i: (0, 0)),
        ],
        out_specs=[
            pl.BlockSpec((1, 128), lambda i: (0, 0)),
            pl.BlockSpec((1, _C), lambda i: (0, 0)),
        ],
        out_shape=[
            jax.ShapeDtypeStruct((1, 128), f32),
            jax.ShapeDtypeStruct((1, _C), f32),
        ],
    )(d, dt, sqc, crow, limr, ccol)

    n = f32(_N)
    recalls = [vec[0, 0] / n, vec[0, 1] / n, vec[0, 2] / n]
    mapr = vec[0, 3] / (n * _R)
    den = jnp.maximum(counts * lim, 1.0)
    r_precision = jnp.mean(num[0] / den)
    return jnp.stack(recalls + [mapr, r_precision])
